# baseline (speedup 1.0000x reference)
import functools

import jax
import jax.numpy as jnp
from jax import lax
from jax.experimental import pallas as pl
from jax.experimental.pallas import tpu as pltpu
from jax.experimental.pallas import tpu_sc as plsc

_NSUB = 16
_LANE = 16
_ROW = 128


def _rsqrt16(d):
    y = jnp.full((_LANE,), 2.0 ** -10, jnp.float32)
    for k in range(9, 0, -1):
        y = jnp.where(d < 4.0 ** k, jnp.float32(2.0 ** -k), y)
    for _ in range(6):
        y = y * (1.5 - 0.5 * d * y * y)
    return y


def _sc_node_scalars(src_e, dst_e, x_pad, npad, eps):
    ch = npad // _NSUB

    def body(src_hbm, dst_hbm, x_hbm, s_hbm,
             src1, dst1, vals, xv, degv, dinvv, uv, pv, sv,
             sh_deg, sh_p, sh_u):
        tid = lax.axis_index("s")
        base = tid * ch

        z16 = jnp.zeros((_LANE,), jnp.float32)

        def zero_body(i, c):
            sv[pl.ds(i * _LANE, _LANE)] = z16
            return c
        lax.fori_loop(0, ch // _LANE, zero_body, 0)
        pltpu.sync_copy(sv, sh_deg.at[pl.ds(base, ch)])
        pltpu.sync_copy(sv, sh_p.at[pl.ds(base, ch)])

        pltpu.sync_copy(src_hbm.at[pl.ds(tid * eps, eps)], src1)
        pltpu.sync_copy(dst_hbm.at[pl.ds(tid * eps, eps)], dst1)
        pltpu.sync_copy(x_hbm.at[pl.ds(base, ch)], xv)

        one16 = jnp.full((_LANE,), 1.0, jnp.float32)

        def ones_body(i, c):
            vals[pl.ds(i * _LANE, _LANE)] = one16
            return c
        lax.fori_loop(0, eps // _LANE, ones_body, 0)
        plsc.subcore_barrier()

        pltpu.sync_copy(vals, sh_deg.at[dst1], add=True)
        plsc.subcore_barrier()

        pltpu.sync_copy(sh_deg.at[pl.ds(base, ch)], degv)

        def du_body(i, c):
            sl = pl.ds(i * _LANE, _LANE)
            d = degv[sl] + 1.0
            y = _rsqrt16(d)
            dinvv[sl] = y
            uv[sl] = xv[sl] * y
            return c
        lax.fori_loop(0, ch // _LANE, du_body, 0)
        pltpu.sync_copy(uv, sh_u.at[pl.ds(base, ch)])
        plsc.subcore_barrier()

        pltpu.sync_copy(sh_u.at[src1], vals)
        pltpu.sync_copy(vals, sh_p.at[dst1], add=True)
        plsc.subcore_barrier()

        pltpu.sync_copy(sh_p.at[pl.ds(base, ch)], pv)

        def s_body(i, c):
            sl = pl.ds(i * _LANE, _LANE)
            sv[sl] = dinvv[sl] * (pv[sl] + uv[sl])
            return c
        lax.fori_loop(0, ch // _LANE, s_body, 0)
        pltpu.sync_copy(sv, s_hbm.at[pl.ds(base, ch)])

    mesh = plsc.VectorSubcoreMesh(
        core_axis_name="c", subcore_axis_name="s", num_cores=1)
    call = pl.kernel(
        body,
        out_type=jax.ShapeDtypeStruct((npad,), jnp.float32),
        mesh=mesh,
        compiler_params=pltpu.CompilerParams(needs_layout_passes=False),
        scratch_types=[
            pltpu.VMEM((eps,), jnp.int32),
            pltpu.VMEM((eps,), jnp.int32),
            pltpu.VMEM((eps,), jnp.float32),
            pltpu.VMEM((ch,), jnp.float32),
            pltpu.VMEM((ch,), jnp.float32),
            pltpu.VMEM((ch,), jnp.float32),
            pltpu.VMEM((ch,), jnp.float32),
            pltpu.VMEM((ch,), jnp.float32),
            pltpu.VMEM((ch,), jnp.float32),
            pltpu.VMEM_SHARED((npad,), jnp.float32),
            pltpu.VMEM_SHARED((npad,), jnp.float32),
            pltpu.VMEM_SHARED((npad,), jnp.float32),
        ],
    )
    return call(src_e, dst_e, x_pad)


def _tc_head(s_pad, batch_pad, n, npad, W1, b1, gamma, beta,
             y_extra, W_l1a, W_l1b, b_l1, W_l2, b_l2, blk):
    g, add = y_extra.shape
    h = W1.shape[1]
    d1 = W_l1a.shape[1]
    d2 = W_l2.shape[1]
    nblk = npad // blk
    inv_sqrt2 = 0.7071067811865476

    def body(s_ref, b_ref, w1_ref, b1_ref, ga_ref, be_ref, ye_ref,
             wa_ref, wb_ref, bl1_ref, wl2_ref, bl2_ref, o_ref,
             gsum, gsq, cnt):
        i = pl.program_id(0)

        @pl.when(i == 0)
        def _init():
            gsum[...] = jnp.zeros_like(gsum)
            gsq[...] = jnp.zeros_like(gsq)
            cnt[...] = jnp.zeros_like(cnt)

        s_blk = s_ref[...]
        agg = s_blk * w1_ref[...] + b1_ref[...]
        hh = 0.5 * agg * (1.0 + lax.erf(agg * inv_sqrt2))

        bi = b_ref[...]
        gid = lax.broadcasted_iota(jnp.int32, (1, g), 1)
        row = lax.broadcasted_iota(jnp.int32, (blk, 1), 0) + i * blk
        oh = jnp.where((bi == gid) & (row < n), 1.0, 0.0)

        dn = (((0,), (0,)), ((), ()))
        gsum[...] += lax.dot_general(oh, hh, dn,
                                     preferred_element_type=jnp.float32)
        gsq[...] += lax.dot_general(oh, hh * hh, dn,
                                    preferred_element_type=jnp.float32)
        cnt[...] += lax.dot_general(oh, jnp.ones((blk, 1), jnp.float32), dn,
                                    preferred_element_type=jnp.float32)

        @pl.when(i == nblk - 1)
        def _fin():
            tot = jnp.sum(gsum[...], axis=0, keepdims=True)
            tot2 = jnp.sum(gsq[...], axis=0, keepdims=True)
            mean = tot / n
            var = tot2 / n - mean * mean
            inv = lax.rsqrt(var + 1e-5)
            c = cnt[...]
            praw = gsum[...] / jnp.maximum(c, 1.0)
            pooled = jnp.where(
                c > 0.0,
                (praw - mean) * inv * ga_ref[...] + be_ref[...],
                0.0)
            z = (jnp.dot(pooled, wa_ref[...],
                         preferred_element_type=jnp.float32)
                 + jnp.dot(ye_ref[...], wb_ref[...],
                           preferred_element_type=jnp.float32)
                 + bl1_ref[...])
            z = jnp.maximum(z, 0.0)
            z2 = (jnp.dot(z, wl2_ref[...],
                          preferred_element_type=jnp.float32)
                  + bl2_ref[...])
            o_ref[...] = 1.0 / (1.0 + jnp.exp(-z2))

    full = lambda shape: pl.BlockSpec(shape, lambda i: (0,) * len(shape))
    return pl.pallas_call(
        body,
        grid=(nblk,),
        in_specs=[
            pl.BlockSpec((blk, 1), lambda i: (i, 0)),
            pl.BlockSpec((blk, 1), lambda i: (i, 0)),
            full((1, h)), full((1, h)), full((1, h)), full((1, h)),
            full((g, add)), full((h, d1)), full((add, d1)), full((1, d1)),
            full((d1, d2)), full((1, d2)),
        ],
        out_specs=full((g, d2)),
        out_shape=jax.ShapeDtypeStruct((g, d2), jnp.float32),
        scratch_shapes=[
            pltpu.VMEM((g, h), jnp.float32),
            pltpu.VMEM((g, h), jnp.float32),
            pltpu.VMEM((g, 1), jnp.float32),
        ],
    )(s_pad, batch_pad, W1, b1.reshape(1, h), gamma.reshape(1, h),
      beta.reshape(1, h), y_extra, W_l1a, W_l1b, b_l1.reshape(1, d1),
      W_l2, b_l2.reshape(1, d2))


def kernel(x, edge_index, batch, y_extra, W1, b1, gamma, beta,
           W_l1, b_l1, W_l2, b_l2):
    n = x.shape[0]
    e = edge_index.shape[1]
    h = W1.shape[1]
    g = y_extra.shape[0]

    unit = _NSUB * _LANE
    npad = ((n + 1 + unit - 1) // unit) * unit
    pad_node = npad - 1

    src = edge_index[0]
    dst = edge_index[1]
    if e % (_NSUB * _LANE) == 0:
        eps = e // _NSUB
    else:
        eps = -(-e // (_NSUB * _LANE)) * _LANE
        pad_e = _NSUB * eps - e
        src = jnp.concatenate([src, jnp.full((pad_e,), pad_node, jnp.int32)])
        dst = jnp.concatenate([dst, jnp.full((pad_e,), pad_node, jnp.int32)])
    x_pad = jnp.concatenate(
        [x[:, 0], jnp.zeros((npad - n,), jnp.float32)])

    s = _sc_node_scalars(src, dst, x_pad, npad, eps)

    batch_pad = jnp.concatenate(
        [batch, jnp.full((npad - n,), g, jnp.int32)]).reshape(npad, 1)
    return _tc_head(s.reshape(npad, 1), batch_pad, n, npad,
                    W1, b1, gamma, beta, y_extra,
                    W_l1[:h], W_l1[h:], b_l1, W_l2, b_l2, blk=npad)


# speedup vs baseline: 1.0030x; 1.0030x over previous
import functools

import jax
import jax.numpy as jnp
from jax import lax
from jax.experimental import pallas as pl
from jax.experimental.pallas import tpu as pltpu
from jax.experimental.pallas import tpu_sc as plsc

_NSUB = 16
_LANE = 16
_ROW = 128


def _rsqrt16(d):
    y = jnp.full((_LANE,), 2.0 ** -10, jnp.float32)
    for k in range(9, 0, -1):
        y = jnp.where(d < 4.0 ** k, jnp.float32(2.0 ** -k), y)
    for _ in range(6):
        y = y * (1.5 - 0.5 * d * y * y)
    return y


def _sc_node_scalars(src_e, dst_e, x_pad, npad, eps):
    ch = npad // _NSUB

    def body(src_hbm, dst_hbm, x_hbm, s_hbm,
             src1, dst1, vals, xv, degv, dinvv, uv, pv, sv,
             sh_deg, sh_p, sh_u):
        tid = lax.axis_index("s")
        base = tid * ch

        z16 = jnp.zeros((_LANE,), jnp.float32)

        def zero_body(i, c):
            sv[pl.ds(i * _LANE, _LANE)] = z16
            return c
        lax.fori_loop(0, ch // _LANE, zero_body, 0)
        pltpu.sync_copy(sv, sh_deg.at[pl.ds(base, ch)])
        pltpu.sync_copy(sv, sh_p.at[pl.ds(base, ch)])

        pltpu.sync_copy(src_hbm.at[pl.ds(tid * eps, eps)], src1)
        pltpu.sync_copy(dst_hbm.at[pl.ds(tid * eps, eps)], dst1)
        pltpu.sync_copy(x_hbm.at[pl.ds(base, ch)], xv)

        one16 = jnp.full((_LANE,), 1.0, jnp.float32)

        def ones_body(i, c):
            vals[pl.ds(i * _LANE, _LANE)] = one16
            return c
        lax.fori_loop(0, eps // _LANE, ones_body, 0)
        plsc.subcore_barrier()

        pltpu.sync_copy(vals, sh_deg.at[dst1], add=True)
        plsc.subcore_barrier()

        pltpu.sync_copy(sh_deg.at[pl.ds(base, ch)], degv)

        def du_body(i, c):
            sl = pl.ds(i * _LANE, _LANE)
            d = degv[sl] + 1.0
            y = _rsqrt16(d)
            dinvv[sl] = y
            uv[sl] = xv[sl] * y
            return c
        lax.fori_loop(0, ch // _LANE, du_body, 0)
        pltpu.sync_copy(uv, sh_u.at[pl.ds(base, ch)])
        plsc.subcore_barrier()

        pltpu.sync_copy(sh_u.at[src1], vals)
        pltpu.sync_copy(vals, sh_p.at[dst1], add=True)
        plsc.subcore_barrier()

        pltpu.sync_copy(sh_p.at[pl.ds(base, ch)], pv)

        def s_body(i, c):
            sl = pl.ds(i * _LANE, _LANE)
            sv[sl] = dinvv[sl] * (pv[sl] + uv[sl])
            return c
        lax.fori_loop(0, ch // _LANE, s_body, 0)
        pltpu.sync_copy(sv, s_hbm.at[pl.ds(base, ch)])

    mesh = plsc.VectorSubcoreMesh(
        core_axis_name="c", subcore_axis_name="s", num_cores=1)
    call = pl.kernel(
        body,
        out_type=jax.ShapeDtypeStruct((npad,), jnp.float32),
        mesh=mesh,
        compiler_params=pltpu.CompilerParams(needs_layout_passes=False),
        scratch_types=[
            pltpu.VMEM((eps,), jnp.int32),
            pltpu.VMEM((eps,), jnp.int32),
            pltpu.VMEM((eps,), jnp.float32),
            pltpu.VMEM((ch,), jnp.float32),
            pltpu.VMEM((ch,), jnp.float32),
            pltpu.VMEM((ch,), jnp.float32),
            pltpu.VMEM((ch,), jnp.float32),
            pltpu.VMEM((ch,), jnp.float32),
            pltpu.VMEM((ch,), jnp.float32),
            pltpu.VMEM_SHARED((npad,), jnp.float32),
            pltpu.VMEM_SHARED((npad,), jnp.float32),
            pltpu.VMEM_SHARED((npad,), jnp.float32),
        ],
    )
    return call(src_e, dst_e, x_pad)


def _tc_head(s_pad, batch_pad, n, npad, W1, b1, gamma, beta,
             y_extra, W_l1a, W_l1b, b_l1, W_l2, b_l2, blk):
    g, add = y_extra.shape
    h = W1.shape[1]
    d1 = W_l1a.shape[1]
    d2 = W_l2.shape[1]
    nblk = npad // blk
    inv_sqrt2 = 0.7071067811865476

    def body(s_ref, b_ref, w1_ref, b1_ref, ga_ref, be_ref, ye_ref,
             wa_ref, wb_ref, bl1_ref, wl2_ref, bl2_ref, o_ref,
             gsum, gsq, cnt):
        i = pl.program_id(0)

        @pl.when(i == 0)
        def _init():
            gsum[...] = jnp.zeros_like(gsum)
            gsq[...] = jnp.zeros_like(gsq)
            cnt[...] = jnp.zeros_like(cnt)

        s_blk = s_ref[...]
        agg = s_blk * w1_ref[...] + b1_ref[...]
        hh = 0.5 * agg * (1.0 + lax.erf(agg * inv_sqrt2))

        bi = b_ref[...]
        gid = lax.broadcasted_iota(jnp.int32, (1, g), 1)
        row = lax.broadcasted_iota(jnp.int32, (blk, 1), 0) + i * blk
        oh = jnp.where((bi == gid) & (row < n), 1.0, 0.0)

        dn = (((0,), (0,)), ((), ()))
        gsum[...] += lax.dot_general(oh, hh, dn,
                                     preferred_element_type=jnp.float32)
        gsq[...] += lax.dot_general(oh, hh * hh, dn,
                                    preferred_element_type=jnp.float32)
        cnt[...] += lax.dot_general(oh, jnp.ones((blk, 1), jnp.float32), dn,
                                    preferred_element_type=jnp.float32)

        @pl.when(i == nblk - 1)
        def _fin():
            tot = jnp.sum(gsum[...], axis=0, keepdims=True)
            tot2 = jnp.sum(gsq[...], axis=0, keepdims=True)
            mean = tot / n
            var = tot2 / n - mean * mean
            inv = lax.rsqrt(var + 1e-5)
            c = cnt[...]
            praw = gsum[...] / jnp.maximum(c, 1.0)
            pooled = jnp.where(
                c > 0.0,
                (praw - mean) * inv * ga_ref[...] + be_ref[...],
                0.0)
            z = (jnp.dot(pooled, wa_ref[...],
                         preferred_element_type=jnp.float32)
                 + jnp.dot(ye_ref[...], wb_ref[...],
                           preferred_element_type=jnp.float32)
                 + bl1_ref[...])
            z = jnp.maximum(z, 0.0)
            z2 = (jnp.dot(z, wl2_ref[...],
                          preferred_element_type=jnp.float32)
                  + bl2_ref[...])
            o_ref[...] = 1.0 / (1.0 + jnp.exp(-z2))

    full = lambda shape: pl.BlockSpec(shape, lambda i: (0,) * len(shape))
    return pl.pallas_call(
        body,
        grid=(nblk,),
        in_specs=[
            pl.BlockSpec((blk, 1), lambda i: (i, 0)),
            pl.BlockSpec((blk, 1), lambda i: (i, 0)),
            full((1, h)), full((1, h)), full((1, h)), full((1, h)),
            full((g, add)), full((h, d1)), full((add, d1)), full((1, d1)),
            full((d1, d2)), full((1, d2)),
        ],
        out_specs=full((g, d2)),
        out_shape=jax.ShapeDtypeStruct((g, d2), jnp.float32),
        scratch_shapes=[
            pltpu.VMEM((g, h), jnp.float32),
            pltpu.VMEM((g, h), jnp.float32),
            pltpu.VMEM((g, 1), jnp.float32),
        ],
    )(s_pad, batch_pad, W1, b1.reshape(1, h), gamma.reshape(1, h),
      beta.reshape(1, h), y_extra, W_l1a, W_l1b, b_l1.reshape(1, d1),
      W_l2, b_l2.reshape(1, d2))


def kernel(x, edge_index, batch, y_extra, W1, b1, gamma, beta,
           W_l1, b_l1, W_l2, b_l2):
    n = x.shape[0]
    e = edge_index.shape[1]
    h = W1.shape[1]
    g = y_extra.shape[0]

    unit = _NSUB * _LANE
    npad = ((n + 1 + unit - 1) // unit) * unit
    pad_node = npad - 1

    src = edge_index[0]
    dst = edge_index[1]
    if e % (_NSUB * _LANE) == 0:
        eps = e // _NSUB
    else:
        eps = -(-e // (_NSUB * _LANE)) * _LANE
        pad_e = _NSUB * eps - e
        src = jnp.concatenate([src, jnp.full((pad_e,), pad_node, jnp.int32)])
        dst = jnp.concatenate([dst, jnp.full((pad_e,), pad_node, jnp.int32)])
    x_pad = jnp.concatenate(
        [x[:, 0], jnp.zeros((npad - n,), jnp.float32)])

    s = _sc_node_scalars(src, dst, x_pad, npad, eps)

    batch_pad = jnp.concatenate(
        [batch, jnp.full((npad - n,), g, jnp.int32)]).reshape(npad, 1)
    return _tc_head(s.reshape(npad, 1), batch_pad, n, npad,
                    W1, b1, gamma, beta, y_extra,
                    W_l1[:h], W_l1[h:], b_l1, W_l2, b_l2, blk=2048)


# speedup vs baseline: 1.0033x; 1.0003x over previous
import functools

import jax
import jax.numpy as jnp
from jax import lax
from jax.experimental import pallas as pl
from jax.experimental.pallas import tpu as pltpu
from jax.experimental.pallas import tpu_sc as plsc

_NSUB = 16
_LANE = 16
_ROW = 128


def _rsqrt16(d):
    y = jnp.full((_LANE,), 2.0 ** -10, jnp.float32)
    for k in range(9, 0, -1):
        y = jnp.where(d < 4.0 ** k, jnp.float32(2.0 ** -k), y)
    for _ in range(6):
        y = y * (1.5 - 0.5 * d * y * y)
    return y


def _sc_node_scalars(src_e, dst_e, x_flat, n, npad, eps):
    ch = npad // _NSUB
    full = n // ch
    rem = n - full * ch

    def body(src_hbm, dst_hbm, x_hbm, s_hbm,
             src1, dst1, vals, xv, degv, dinvv, uv, pv, sv,
             sh_deg, sh_p, sh_u):
        tid = lax.axis_index("s")
        base = tid * ch

        z16 = jnp.zeros((_LANE,), jnp.float32)

        def zero_body(i, c):
            sv[pl.ds(i * _LANE, _LANE)] = z16
            xv[pl.ds(i * _LANE, _LANE)] = z16
            return c
        lax.fori_loop(0, ch // _LANE, zero_body, 0)
        pltpu.sync_copy(sv, sh_deg.at[pl.ds(base, ch)])
        pltpu.sync_copy(sv, sh_p.at[pl.ds(base, ch)])

        pltpu.sync_copy(src_hbm.at[pl.ds(tid * eps, eps)], src1)
        pltpu.sync_copy(dst_hbm.at[pl.ds(tid * eps, eps)], dst1)
        if full > 0:
            @pl.when(tid < full)
            def _copy_full():
                pltpu.sync_copy(x_hbm.at[pl.ds(base, ch)], xv)
        if rem > 0:
            @pl.when(tid == full)
            def _copy_rem():
                pltpu.sync_copy(x_hbm.at[pl.ds(full * ch, rem)],
                                xv.at[pl.ds(0, rem)])

        one16 = jnp.full((_LANE,), 1.0, jnp.float32)

        def ones_body(i, c):
            vals[pl.ds(i * _LANE, _LANE)] = one16
            return c
        lax.fori_loop(0, eps // _LANE, ones_body, 0)
        plsc.subcore_barrier()

        pltpu.sync_copy(vals, sh_deg.at[dst1], add=True)
        plsc.subcore_barrier()

        pltpu.sync_copy(sh_deg.at[pl.ds(base, ch)], degv)

        def du_body(i, c):
            sl = pl.ds(i * _LANE, _LANE)
            d = degv[sl] + 1.0
            y = _rsqrt16(d)
            dinvv[sl] = y
            uv[sl] = xv[sl] * y
            return c
        lax.fori_loop(0, ch // _LANE, du_body, 0)
        pltpu.sync_copy(uv, sh_u.at[pl.ds(base, ch)])
        plsc.subcore_barrier()

        pltpu.sync_copy(sh_u.at[src1], vals)
        pltpu.sync_copy(vals, sh_p.at[dst1], add=True)
        plsc.subcore_barrier()

        pltpu.sync_copy(sh_p.at[pl.ds(base, ch)], pv)

        def s_body(i, c):
            sl = pl.ds(i * _LANE, _LANE)
            sv[sl] = dinvv[sl] * (pv[sl] + uv[sl])
            return c
        lax.fori_loop(0, ch // _LANE, s_body, 0)
        pltpu.sync_copy(sv, s_hbm.at[pl.ds(base, ch)])

    mesh = plsc.VectorSubcoreMesh(
        core_axis_name="c", subcore_axis_name="s", num_cores=1)
    call = pl.kernel(
        body,
        out_type=jax.ShapeDtypeStruct((npad,), jnp.float32),
        mesh=mesh,
        compiler_params=pltpu.CompilerParams(needs_layout_passes=False),
        scratch_types=[
            pltpu.VMEM((eps,), jnp.int32),
            pltpu.VMEM((eps,), jnp.int32),
            pltpu.VMEM((eps,), jnp.float32),
            pltpu.VMEM((ch,), jnp.float32),
            pltpu.VMEM((ch,), jnp.float32),
            pltpu.VMEM((ch,), jnp.float32),
            pltpu.VMEM((ch,), jnp.float32),
            pltpu.VMEM((ch,), jnp.float32),
            pltpu.VMEM((ch,), jnp.float32),
            pltpu.VMEM_SHARED((npad,), jnp.float32),
            pltpu.VMEM_SHARED((npad,), jnp.float32),
            pltpu.VMEM_SHARED((npad,), jnp.float32),
        ],
    )
    return call(src_e, dst_e, x_flat)


def _tc_head(s_pad, batch, n, npad, W1, b1, gamma, beta,
             y_extra, W_l1a, W_l1b, b_l1, W_l2, b_l2):
    g, add = y_extra.shape
    h = W1.shape[1]
    d1 = W_l1a.shape[1]
    d2 = W_l2.shape[1]
    inv_sqrt2 = 0.7071067811865476

    def body(s_ref, b_ref, w1_ref, b1_ref, ga_ref, be_ref, ye_ref,
             wa_ref, wb_ref, bl1_ref, wl2_ref, bl2_ref, o_ref):
        s_blk = s_ref[...]
        agg = s_blk * w1_ref[...] + b1_ref[...]
        hh = 0.5 * agg * (1.0 + lax.erf(agg * inv_sqrt2))

        bi = b_ref[...]
        gid = lax.broadcasted_iota(jnp.int32, (1, g), 1)
        row = lax.broadcasted_iota(jnp.int32, (npad, 1), 0)
        valid = row < n
        oh = jnp.where((bi == gid) & valid, 1.0, 0.0)

        dn = (((0,), (0,)), ((), ()))
        gsum = lax.dot_general(oh, hh, dn,
                               preferred_element_type=jnp.float32)
        gsq = lax.dot_general(oh, hh * hh, dn,
                              preferred_element_type=jnp.float32)
        cnt = lax.dot_general(oh, jnp.ones((npad, 1), jnp.float32), dn,
                              preferred_element_type=jnp.float32)

        tot = jnp.sum(gsum, axis=0, keepdims=True)
        tot2 = jnp.sum(gsq, axis=0, keepdims=True)
        mean = tot / n
        var = tot2 / n - mean * mean
        inv = lax.rsqrt(var + 1e-5)
        praw = gsum / jnp.maximum(cnt, 1.0)
        pooled = jnp.where(
            cnt > 0.0,
            (praw - mean) * inv * ga_ref[...] + be_ref[...],
            0.0)
        z = (jnp.dot(pooled, wa_ref[...],
                     preferred_element_type=jnp.float32)
             + jnp.dot(ye_ref[...], wb_ref[...],
                       preferred_element_type=jnp.float32)
             + bl1_ref[...])
        z = jnp.maximum(z, 0.0)
        z2 = (jnp.dot(z, wl2_ref[...],
                      preferred_element_type=jnp.float32)
              + bl2_ref[...])
        o_ref[...] = 1.0 / (1.0 + jnp.exp(-z2))

    full = lambda shape: pl.BlockSpec(shape, lambda i: (0,) * len(shape))
    return pl.pallas_call(
        body,
        grid=(1,),
        in_specs=[
            full((npad, 1)),
            full((npad, 1)),
            full((1, h)), full((1, h)), full((1, h)), full((1, h)),
            full((g, add)), full((h, d1)), full((add, d1)), full((1, d1)),
            full((d1, d2)), full((1, d2)),
        ],
        out_specs=full((g, d2)),
        out_shape=jax.ShapeDtypeStruct((g, d2), jnp.float32),
    )(s_pad, batch, W1, b1.reshape(1, h), gamma.reshape(1, h),
      beta.reshape(1, h), y_extra, W_l1a, W_l1b, b_l1.reshape(1, d1),
      W_l2, b_l2.reshape(1, d2))


def kernel(x, edge_index, batch, y_extra, W1, b1, gamma, beta,
           W_l1, b_l1, W_l2, b_l2):
    n = x.shape[0]
    e = edge_index.shape[1]
    h = W1.shape[1]
    g = y_extra.shape[0]

    unit = _NSUB * _LANE
    npad = ((n + 1 + unit - 1) // unit) * unit
    pad_node = npad - 1

    src = edge_index[0]
    dst = edge_index[1]
    if e % (_NSUB * _LANE) == 0:
        eps = e // _NSUB
    else:
        eps = -(-e // (_NSUB * _LANE)) * _LANE
        pad_e = _NSUB * eps - e
        src = jnp.concatenate([src, jnp.full((pad_e,), pad_node, jnp.int32)])
        dst = jnp.concatenate([dst, jnp.full((pad_e,), pad_node, jnp.int32)])

    x_flat = x[:, 0]
    ch = npad // _NSUB
    if (n % ch) % _LANE != 0:
        x_flat = jnp.concatenate(
            [x_flat, jnp.zeros((npad - n,), jnp.float32)])
        s = _sc_node_scalars(src, dst, x_flat, npad, npad, eps)
    else:
        s = _sc_node_scalars(src, dst, x_flat, n, npad, eps)

    return _tc_head(s.reshape(npad, 1), batch.reshape(n, 1), n, npad,
                    W1, b1, gamma, beta, y_extra,
                    W_l1[:h], W_l1[h:], b_l1, W_l2, b_l2)


# speedup vs baseline: 1.1584x; 1.1546x over previous
import functools

import jax
import jax.numpy as jnp
from jax import lax
from jax.experimental import pallas as pl
from jax.experimental.pallas import tpu as pltpu
from jax.experimental.pallas import tpu_sc as plsc

_NSUB = 16
_LANE = 16
_ROW = 128


def _rsqrt16(d):
    y = jnp.full((_LANE,), 2.0 ** -10, jnp.float32)
    for k in range(9, 0, -1):
        y = jnp.where(d < 4.0 ** k, jnp.float32(2.0 ** -k), y)
    for _ in range(6):
        y = y * (1.5 - 0.5 * d * y * y)
    return y


def _sc_node_scalars(edge_ix, x_flat, n, npad, eps):
    ch = npad // _NSUB
    full = n // ch
    rem = n - full * ch

    def body(ei_hbm, x_hbm, s_hbm,
             src1, dst1, vals, xv, degv, dinvv, uv, pv, sv,
             sh_deg, sh_p, sh_u):
        tid = lax.axis_index("s")
        base = tid * ch

        z16 = jnp.zeros((_LANE,), jnp.float32)

        def zero_body(i, c):
            sv[pl.ds(i * _LANE, _LANE)] = z16
            xv[pl.ds(i * _LANE, _LANE)] = z16
            return c
        lax.fori_loop(0, ch // _LANE, zero_body, 0)
        pltpu.sync_copy(sv, sh_deg.at[pl.ds(base, ch)])
        pltpu.sync_copy(sv, sh_p.at[pl.ds(base, ch)])

        pltpu.sync_copy(ei_hbm.at[pl.ds(tid * eps, eps)], src1)
        pltpu.sync_copy(ei_hbm.at[pl.ds(_NSUB * eps + tid * eps, eps)], dst1)
        if full > 0:
            @pl.when(tid < full)
            def _copy_full():
                pltpu.sync_copy(x_hbm.at[pl.ds(base, ch)], xv)
        if rem > 0:
            @pl.when(tid == full)
            def _copy_rem():
                pltpu.sync_copy(x_hbm.at[pl.ds(full * ch, rem)],
                                xv.at[pl.ds(0, rem)])

        one16 = jnp.full((_LANE,), 1.0, jnp.float32)

        def ones_body(i, c):
            vals[pl.ds(i * _LANE, _LANE)] = one16
            return c
        lax.fori_loop(0, eps // _LANE, ones_body, 0)
        plsc.subcore_barrier()

        pltpu.sync_copy(vals, sh_deg.at[dst1], add=True)
        plsc.subcore_barrier()

        pltpu.sync_copy(sh_deg.at[pl.ds(base, ch)], degv)

        def du_body(i, c):
            sl = pl.ds(i * _LANE, _LANE)
            d = degv[sl] + 1.0
            y = _rsqrt16(d)
            dinvv[sl] = y
            uv[sl] = xv[sl] * y
            return c
        lax.fori_loop(0, ch // _LANE, du_body, 0)
        pltpu.sync_copy(uv, sh_u.at[pl.ds(base, ch)])
        plsc.subcore_barrier()

        pltpu.sync_copy(sh_u.at[src1], vals)
        pltpu.sync_copy(vals, sh_p.at[dst1], add=True)
        plsc.subcore_barrier()

        pltpu.sync_copy(sh_p.at[pl.ds(base, ch)], pv)

        def s_body(i, c):
            sl = pl.ds(i * _LANE, _LANE)
            sv[sl] = dinvv[sl] * (pv[sl] + uv[sl])
            return c
        lax.fori_loop(0, ch // _LANE, s_body, 0)
        pltpu.sync_copy(sv, s_hbm.at[pl.ds(base, ch)])

    mesh = plsc.VectorSubcoreMesh(
        core_axis_name="c", subcore_axis_name="s", num_cores=1)
    call = pl.kernel(
        body,
        out_type=jax.ShapeDtypeStruct((npad,), jnp.float32),
        mesh=mesh,
        compiler_params=pltpu.CompilerParams(needs_layout_passes=False),
        scratch_types=[
            pltpu.VMEM((eps,), jnp.int32),
            pltpu.VMEM((eps,), jnp.int32),
            pltpu.VMEM((eps,), jnp.float32),
            pltpu.VMEM((ch,), jnp.float32),
            pltpu.VMEM((ch,), jnp.float32),
            pltpu.VMEM((ch,), jnp.float32),
            pltpu.VMEM((ch,), jnp.float32),
            pltpu.VMEM((ch,), jnp.float32),
            pltpu.VMEM((ch,), jnp.float32),
            pltpu.VMEM_SHARED((npad,), jnp.float32),
            pltpu.VMEM_SHARED((npad,), jnp.float32),
            pltpu.VMEM_SHARED((npad,), jnp.float32),
        ],
    )
    return call(edge_ix, x_flat)


def _tc_head(s_pad, batch, n, npad, W1, b1, gamma, beta,
             y_extra, W_l1a, W_l1b, b_l1, W_l2, b_l2):
    g, add = y_extra.shape
    h = W1.shape[1]
    d1 = W_l1a.shape[1]
    d2 = W_l2.shape[1]
    inv_sqrt2 = 0.7071067811865476

    def body(s_ref, b_ref, w1_ref, b1_ref, ga_ref, be_ref, ye_ref,
             wa_ref, wb_ref, bl1_ref, wl2_ref, bl2_ref, o_ref):
        s_blk = s_ref[...]
        agg = s_blk * w1_ref[...] + b1_ref[...]
        hh = 0.5 * agg * (1.0 + lax.erf(agg * inv_sqrt2))

        bi = b_ref[...]
        gid = lax.broadcasted_iota(jnp.int32, (1, g), 1)
        row = lax.broadcasted_iota(jnp.int32, (npad, 1), 0)
        valid = row < n
        oh = jnp.where((bi == gid) & valid, 1.0, 0.0)

        dn = (((0,), (0,)), ((), ()))
        gsum = lax.dot_general(oh, hh, dn,
                               preferred_element_type=jnp.float32)
        gsq = lax.dot_general(oh, hh * hh, dn,
                              preferred_element_type=jnp.float32)
        cnt = lax.dot_general(oh, jnp.ones((npad, 1), jnp.float32), dn,
                              preferred_element_type=jnp.float32)

        tot = jnp.sum(gsum, axis=0, keepdims=True)
        tot2 = jnp.sum(gsq, axis=0, keepdims=True)
        mean = tot / n
        var = tot2 / n - mean * mean
        inv = lax.rsqrt(var + 1e-5)
        praw = gsum / jnp.maximum(cnt, 1.0)
        pooled = jnp.where(
            cnt > 0.0,
            (praw - mean) * inv * ga_ref[...] + be_ref[...],
            0.0)
        z = (jnp.dot(pooled, wa_ref[...],
                     preferred_element_type=jnp.float32)
             + jnp.dot(ye_ref[...], wb_ref[...],
                       preferred_element_type=jnp.float32)
             + bl1_ref[...])
        z = jnp.maximum(z, 0.0)
        z2 = (jnp.dot(z, wl2_ref[...],
                      preferred_element_type=jnp.float32)
              + bl2_ref[...])
        o_ref[...] = 1.0 / (1.0 + jnp.exp(-z2))

    full = lambda shape: pl.BlockSpec(shape, lambda i: (0,) * len(shape))
    return pl.pallas_call(
        body,
        grid=(1,),
        in_specs=[
            full((npad, 1)),
            full((npad, 1)),
            full((1, h)), full((1, h)), full((1, h)), full((1, h)),
            full((g, add)), full((h, d1)), full((add, d1)), full((1, d1)),
            full((d1, d2)), full((1, d2)),
        ],
        out_specs=full((g, d2)),
        out_shape=jax.ShapeDtypeStruct((g, d2), jnp.float32),
    )(s_pad, batch, W1, b1.reshape(1, h), gamma.reshape(1, h),
      beta.reshape(1, h), y_extra, W_l1a, W_l1b, b_l1.reshape(1, d1),
      W_l2, b_l2.reshape(1, d2))


def kernel(x, edge_index, batch, y_extra, W1, b1, gamma, beta,
           W_l1, b_l1, W_l2, b_l2):
    n = x.shape[0]
    e = edge_index.shape[1]
    h = W1.shape[1]
    g = y_extra.shape[0]

    unit = _NSUB * _LANE
    npad = ((n + 1 + unit - 1) // unit) * unit
    pad_node = npad - 1

    ei = edge_index
    if e % (_NSUB * _LANE) == 0:
        eps = e // _NSUB
    else:
        eps = -(-e // (_NSUB * _LANE)) * _LANE
        pad_e = _NSUB * eps - e
        ei = jnp.concatenate(
            [ei, jnp.full((2, pad_e), pad_node, jnp.int32)], axis=1)
    ei = ei.reshape(-1)

    x_flat = x[:, 0]
    ch = npad // _NSUB
    if (n % ch) % _LANE != 0:
        x_flat = jnp.concatenate(
            [x_flat, jnp.zeros((npad - n,), jnp.float32)])
        s = _sc_node_scalars(ei, x_flat, npad, npad, eps)
    else:
        s = _sc_node_scalars(ei, x_flat, n, npad, eps)

    return _tc_head(s.reshape(npad, 1), batch.reshape(n, 1), n, npad,
                    W1, b1, gamma, beta, y_extra,
                    W_l1[:h], W_l1[h:], b_l1, W_l2, b_l2)


# speedup vs baseline: 1.2124x; 1.0466x over previous
import functools

import jax
import jax.numpy as jnp
from jax import lax
from jax.experimental import pallas as pl
from jax.experimental.pallas import tpu as pltpu
from jax.experimental.pallas import tpu_sc as plsc

_NSUB = 16
_LANE = 16
_ROW = 128


def _rsqrt16(d):
    y = jnp.full((_LANE,), 2.0 ** -10, jnp.float32)
    for k in range(9, 0, -1):
        y = jnp.where(d < 4.0 ** k, jnp.float32(2.0 ** -k), y)
    for _ in range(6):
        y = y * (1.5 - 0.5 * d * y * y)
    return y


def _sc_node_scalars(edge_ix, x_flat, n, npad, eps):
    ch = npad // _NSUB
    full = n // ch
    rem = n - full * ch

    def body(ei_hbm, x_hbm, ones_hbm, s_hbm,
             src1, dst1, vals, xv, degv, dinvv, uv, pv, sv,
             sh_deg, sh_p, sh_u):
        tid = lax.axis_index("s")
        base = tid * ch

        z16 = jnp.zeros((_LANE,), jnp.float32)

        def zero_body(i, c):
            sv[pl.ds(i * _LANE, _LANE)] = z16
            xv[pl.ds(i * _LANE, _LANE)] = z16
            return c
        lax.fori_loop(0, ch // _LANE, zero_body, 0)
        pltpu.sync_copy(sv, sh_deg.at[pl.ds(base, ch)])
        pltpu.sync_copy(sv, sh_p.at[pl.ds(base, ch)])

        pltpu.sync_copy(ei_hbm.at[pl.ds(tid * eps, eps)], src1)
        pltpu.sync_copy(ei_hbm.at[pl.ds(_NSUB * eps + tid * eps, eps)], dst1)
        if full > 0:
            @pl.when(tid < full)
            def _copy_full():
                pltpu.sync_copy(x_hbm.at[pl.ds(base, ch)], xv)
        if rem > 0:
            @pl.when(tid == full)
            def _copy_rem():
                pltpu.sync_copy(x_hbm.at[pl.ds(full * ch, rem)],
                                xv.at[pl.ds(0, rem)])

        pltpu.sync_copy(ones_hbm, vals)
        plsc.subcore_barrier()

        pltpu.sync_copy(vals, sh_deg.at[dst1], add=True)
        plsc.subcore_barrier()

        pltpu.sync_copy(sh_deg.at[pl.ds(base, ch)], degv)

        def du_body(i, c):
            sl = pl.ds(i * _LANE, _LANE)
            d = degv[sl] + 1.0
            y = _rsqrt16(d)
            dinvv[sl] = y
            uv[sl] = xv[sl] * y
            return c
        lax.fori_loop(0, ch // _LANE, du_body, 0)
        pltpu.sync_copy(uv, sh_u.at[pl.ds(base, ch)])
        plsc.subcore_barrier()

        pltpu.sync_copy(sh_u.at[src1], vals)
        pltpu.sync_copy(vals, sh_p.at[dst1], add=True)
        plsc.subcore_barrier()

        pltpu.sync_copy(sh_p.at[pl.ds(base, ch)], pv)

        def s_body(i, c):
            sl = pl.ds(i * _LANE, _LANE)
            sv[sl] = dinvv[sl] * (pv[sl] + uv[sl])
            return c
        lax.fori_loop(0, ch // _LANE, s_body, 0)
        pltpu.sync_copy(sv, s_hbm.at[pl.ds(base, ch)])

    mesh = plsc.VectorSubcoreMesh(
        core_axis_name="c", subcore_axis_name="s", num_cores=1)
    call = pl.kernel(
        body,
        out_type=jax.ShapeDtypeStruct((npad,), jnp.float32),
        mesh=mesh,
        compiler_params=pltpu.CompilerParams(needs_layout_passes=False),
        scratch_types=[
            pltpu.VMEM((eps,), jnp.int32),
            pltpu.VMEM((eps,), jnp.int32),
            pltpu.VMEM((eps,), jnp.float32),
            pltpu.VMEM((ch,), jnp.float32),
            pltpu.VMEM((ch,), jnp.float32),
            pltpu.VMEM((ch,), jnp.float32),
            pltpu.VMEM((ch,), jnp.float32),
            pltpu.VMEM((ch,), jnp.float32),
            pltpu.VMEM((ch,), jnp.float32),
            pltpu.VMEM_SHARED((npad,), jnp.float32),
            pltpu.VMEM_SHARED((npad,), jnp.float32),
            pltpu.VMEM_SHARED((npad,), jnp.float32),
        ],
    )
    return call(edge_ix, x_flat, jnp.ones((eps,), jnp.float32))


def _tc_head(s_pad, batch, n, npad, W1, b1, gamma, beta,
             y_extra, W_l1a, W_l1b, b_l1, W_l2, b_l2):
    g, add = y_extra.shape
    h = W1.shape[1]
    d1 = W_l1a.shape[1]
    d2 = W_l2.shape[1]
    inv_sqrt2 = 0.7071067811865476

    def body(s_ref, b_ref, w1_ref, b1_ref, ga_ref, be_ref, ye_ref,
             wa_ref, wb_ref, bl1_ref, wl2_ref, bl2_ref, o_ref):
        s_blk = s_ref[...]
        agg = s_blk * w1_ref[...] + b1_ref[...]
        hh = 0.5 * agg * (1.0 + lax.erf(agg * inv_sqrt2))

        bi = b_ref[...]
        gid = lax.broadcasted_iota(jnp.int32, (1, g), 1)
        row = lax.broadcasted_iota(jnp.int32, (npad, 1), 0)
        valid = row < n
        oh = jnp.where((bi == gid) & valid, 1.0, 0.0)

        dn = (((0,), (0,)), ((), ()))
        gsum = lax.dot_general(oh, hh, dn,
                               preferred_element_type=jnp.float32)
        gsq = lax.dot_general(oh, hh * hh, dn,
                              preferred_element_type=jnp.float32)
        cnt = lax.dot_general(oh, jnp.ones((npad, 1), jnp.float32), dn,
                              preferred_element_type=jnp.float32)

        tot = jnp.sum(gsum, axis=0, keepdims=True)
        tot2 = jnp.sum(gsq, axis=0, keepdims=True)
        mean = tot / n
        var = tot2 / n - mean * mean
        inv = lax.rsqrt(var + 1e-5)
        praw = gsum / jnp.maximum(cnt, 1.0)
        pooled = jnp.where(
            cnt > 0.0,
            (praw - mean) * inv * ga_ref[...] + be_ref[...],
            0.0)
        z = (jnp.dot(pooled, wa_ref[...],
                     preferred_element_type=jnp.float32)
             + jnp.dot(ye_ref[...], wb_ref[...],
                       preferred_element_type=jnp.float32)
             + bl1_ref[...])
        z = jnp.maximum(z, 0.0)
        z2 = (jnp.dot(z, wl2_ref[...],
                      preferred_element_type=jnp.float32)
              + bl2_ref[...])
        o_ref[...] = 1.0 / (1.0 + jnp.exp(-z2))

    full = lambda shape: pl.BlockSpec(shape, lambda i: (0,) * len(shape))
    return pl.pallas_call(
        body,
        grid=(1,),
        in_specs=[
            full((npad, 1)),
            full((npad, 1)),
            full((1, h)), full((1, h)), full((1, h)), full((1, h)),
            full((g, add)), full((h, d1)), full((add, d1)), full((1, d1)),
            full((d1, d2)), full((1, d2)),
        ],
        out_specs=full((g, d2)),
        out_shape=jax.ShapeDtypeStruct((g, d2), jnp.float32),
    )(s_pad, batch, W1, b1.reshape(1, h), gamma.reshape(1, h),
      beta.reshape(1, h), y_extra, W_l1a, W_l1b, b_l1.reshape(1, d1),
      W_l2, b_l2.reshape(1, d2))


def kernel(x, edge_index, batch, y_extra, W1, b1, gamma, beta,
           W_l1, b_l1, W_l2, b_l2):
    n = x.shape[0]
    e = edge_index.shape[1]
    h = W1.shape[1]
    g = y_extra.shape[0]

    unit = _NSUB * _LANE
    npad = ((n + 1 + unit - 1) // unit) * unit
    pad_node = npad - 1

    ei = edge_index
    if e % (_NSUB * _LANE) == 0:
        eps = e // _NSUB
    else:
        eps = -(-e // (_NSUB * _LANE)) * _LANE
        pad_e = _NSUB * eps - e
        ei = jnp.concatenate(
            [ei, jnp.full((2, pad_e), pad_node, jnp.int32)], axis=1)
    ei = ei.reshape(-1)

    x_flat = x[:, 0]
    ch = npad // _NSUB
    if (n % ch) % _LANE != 0:
        x_flat = jnp.concatenate(
            [x_flat, jnp.zeros((npad - n,), jnp.float32)])
        s = _sc_node_scalars(ei, x_flat, npad, npad, eps)
    else:
        s = _sc_node_scalars(ei, x_flat, n, npad, eps)

    return _tc_head(s.reshape(npad, 1), batch.reshape(n, 1), n, npad,
                    W1, b1, gamma, beta, y_extra,
                    W_l1[:h], W_l1[h:], b_l1, W_l2, b_l2)


# speedup vs baseline: 1.2379x; 1.0210x over previous
import functools

import jax
import jax.numpy as jnp
from jax import lax
from jax.experimental import pallas as pl
from jax.experimental.pallas import tpu as pltpu
from jax.experimental.pallas import tpu_sc as plsc

_NSUB = 16
_LANE = 16
_ROW = 128


def _rsqrt16(d):
    y = jnp.full((_LANE,), 2.0 ** -10, jnp.float32)
    for k in range(9, 0, -1):
        y = jnp.where(d < 4.0 ** k, jnp.float32(2.0 ** -k), y)
    for _ in range(6):
        y = y * (1.5 - 0.5 * d * y * y)
    return y


def _sc_node_scalars(edge_ix, x_flat, n, npad, eps):
    ch = npad // _NSUB
    full = n // ch
    rem = n - full * ch

    def body(ei_hbm, x_hbm, ones_hbm, s_hbm,
             src1, dst1, vals, xv, degv, dinvv, uv, pv, sv,
             sh_deg, sh_p, sh_u, sem_d, sem_o, sem_s, sem_x):
        tid = lax.axis_index("s")
        base = tid * ch

        c_dst = pltpu.async_copy(
            ei_hbm.at[pl.ds(_NSUB * eps + tid * eps, eps)], dst1, sem_d)
        c_one = pltpu.async_copy(ones_hbm, vals, sem_o)
        c_src = pltpu.async_copy(
            ei_hbm.at[pl.ds(tid * eps, eps)], src1, sem_s)

        z16 = jnp.zeros((_LANE,), jnp.float32)
        if full > 0:
            @pl.when(tid < full)
            def _copy_full():
                pltpu.async_copy(x_hbm.at[pl.ds(base, ch)], xv, sem_x)

        @pl.when(tid >= full)
        def _zero_x():
            def zx(i, c):
                xv[pl.ds(i * _LANE, _LANE)] = z16
                return c
            lax.fori_loop(0, ch // _LANE, zx, 0)

        def zero_body(i, c):
            sv[pl.ds(i * _LANE, _LANE)] = z16
            return c
        lax.fori_loop(0, ch // _LANE, zero_body, 0)
        pltpu.sync_copy(sv, sh_deg.at[pl.ds(base, ch)])
        pltpu.sync_copy(sv, sh_p.at[pl.ds(base, ch)])

        if rem > 0:
            @pl.when(tid == full)
            def _copy_rem():
                pltpu.async_copy(x_hbm.at[pl.ds(full * ch, rem)],
                                 xv.at[pl.ds(0, rem)], sem_x)

        c_dst.wait()
        c_one.wait()
        plsc.subcore_barrier()

        pltpu.sync_copy(vals, sh_deg.at[dst1], add=True)
        plsc.subcore_barrier()

        pltpu.sync_copy(sh_deg.at[pl.ds(base, ch)], degv)
        if full > 0:
            @pl.when(tid < full)
            def _wait_xf():
                pltpu.make_async_copy(
                    x_hbm.at[pl.ds(base, ch)], xv, sem_x).wait()
        if rem > 0:
            @pl.when(tid == full)
            def _wait_xr():
                pltpu.make_async_copy(
                    x_hbm.at[pl.ds(full * ch, rem)],
                    xv.at[pl.ds(0, rem)], sem_x).wait()

        def du_body(i, c):
            sl = pl.ds(i * _LANE, _LANE)
            d = degv[sl] + 1.0
            y = _rsqrt16(d)
            dinvv[sl] = y
            uv[sl] = xv[sl] * y
            return c
        lax.fori_loop(0, ch // _LANE, du_body, 0)
        pltpu.sync_copy(uv, sh_u.at[pl.ds(base, ch)])
        plsc.subcore_barrier()

        c_src.wait()
        pltpu.sync_copy(sh_u.at[src1], vals)
        pltpu.sync_copy(vals, sh_p.at[dst1], add=True)
        plsc.subcore_barrier()

        pltpu.sync_copy(sh_p.at[pl.ds(base, ch)], pv)

        def s_body(i, c):
            sl = pl.ds(i * _LANE, _LANE)
            sv[sl] = dinvv[sl] * (pv[sl] + uv[sl])
            return c
        lax.fori_loop(0, ch // _LANE, s_body, 0)
        pltpu.sync_copy(sv, s_hbm.at[pl.ds(base, ch)])

    mesh = plsc.VectorSubcoreMesh(
        core_axis_name="c", subcore_axis_name="s", num_cores=1)
    call = pl.kernel(
        body,
        out_type=jax.ShapeDtypeStruct((npad,), jnp.float32),
        mesh=mesh,
        compiler_params=pltpu.CompilerParams(needs_layout_passes=False),
        scratch_types=[
            pltpu.VMEM((eps,), jnp.int32),
            pltpu.VMEM((eps,), jnp.int32),
            pltpu.VMEM((eps,), jnp.float32),
            pltpu.VMEM((ch,), jnp.float32),
            pltpu.VMEM((ch,), jnp.float32),
            pltpu.VMEM((ch,), jnp.float32),
            pltpu.VMEM((ch,), jnp.float32),
            pltpu.VMEM((ch,), jnp.float32),
            pltpu.VMEM((ch,), jnp.float32),
            pltpu.VMEM_SHARED((npad,), jnp.float32),
            pltpu.VMEM_SHARED((npad,), jnp.float32),
            pltpu.VMEM_SHARED((npad,), jnp.float32),
            pltpu.SemaphoreType.DMA,
            pltpu.SemaphoreType.DMA,
            pltpu.SemaphoreType.DMA,
            pltpu.SemaphoreType.DMA,
        ],
    )
    return call(edge_ix, x_flat, jnp.ones((eps,), jnp.float32))


def _tc_head(s_pad, batch, n, npad, W1, b1, gamma, beta,
             y_extra, W_l1a, W_l1b, b_l1, W_l2, b_l2):
    g, add = y_extra.shape
    h = W1.shape[1]
    d1 = W_l1a.shape[1]
    d2 = W_l2.shape[1]
    inv_sqrt2 = 0.7071067811865476

    def body(s_ref, b_ref, w1_ref, b1_ref, ga_ref, be_ref, ye_ref,
             wa_ref, wb_ref, bl1_ref, wl2_ref, bl2_ref, o_ref):
        s_blk = s_ref[...]
        agg = s_blk * w1_ref[...] + b1_ref[...]
        hh = 0.5 * agg * (1.0 + lax.erf(agg * inv_sqrt2))

        bi = b_ref[...]
        gid = lax.broadcasted_iota(jnp.int32, (1, g), 1)
        row = lax.broadcasted_iota(jnp.int32, (npad, 1), 0)
        valid = row < n
        oh = jnp.where((bi == gid) & valid, 1.0, 0.0)

        dn = (((0,), (0,)), ((), ()))
        gsum = lax.dot_general(oh, hh, dn,
                               preferred_element_type=jnp.float32)
        gsq = lax.dot_general(oh, hh * hh, dn,
                              preferred_element_type=jnp.float32)
        cnt = lax.dot_general(oh, jnp.ones((npad, 1), jnp.float32), dn,
                              preferred_element_type=jnp.float32)

        tot = jnp.sum(gsum, axis=0, keepdims=True)
        tot2 = jnp.sum(gsq, axis=0, keepdims=True)
        mean = tot / n
        var = tot2 / n - mean * mean
        inv = lax.rsqrt(var + 1e-5)
        praw = gsum / jnp.maximum(cnt, 1.0)
        pooled = jnp.where(
            cnt > 0.0,
            (praw - mean) * inv * ga_ref[...] + be_ref[...],
            0.0)
        z = (jnp.dot(pooled, wa_ref[...],
                     preferred_element_type=jnp.float32)
             + jnp.dot(ye_ref[...], wb_ref[...],
                       preferred_element_type=jnp.float32)
             + bl1_ref[...])
        z = jnp.maximum(z, 0.0)
        z2 = (jnp.dot(z, wl2_ref[...],
                      preferred_element_type=jnp.float32)
              + bl2_ref[...])
        o_ref[...] = 1.0 / (1.0 + jnp.exp(-z2))

    full = lambda shape: pl.BlockSpec(shape, lambda i: (0,) * len(shape))
    return pl.pallas_call(
        body,
        grid=(1,),
        in_specs=[
            full((npad, 1)),
            full((npad, 1)),
            full((1, h)), full((1, h)), full((1, h)), full((1, h)),
            full((g, add)), full((h, d1)), full((add, d1)), full((1, d1)),
            full((d1, d2)), full((1, d2)),
        ],
        out_specs=full((g, d2)),
        out_shape=jax.ShapeDtypeStruct((g, d2), jnp.float32),
    )(s_pad, batch, W1, b1.reshape(1, h), gamma.reshape(1, h),
      beta.reshape(1, h), y_extra, W_l1a, W_l1b, b_l1.reshape(1, d1),
      W_l2, b_l2.reshape(1, d2))


def kernel(x, edge_index, batch, y_extra, W1, b1, gamma, beta,
           W_l1, b_l1, W_l2, b_l2):
    n = x.shape[0]
    e = edge_index.shape[1]
    h = W1.shape[1]
    g = y_extra.shape[0]

    unit = _NSUB * _LANE
    npad = ((n + 1 + unit - 1) // unit) * unit
    pad_node = npad - 1

    ei = edge_index
    if e % (_NSUB * _LANE) == 0:
        eps = e // _NSUB
    else:
        eps = -(-e // (_NSUB * _LANE)) * _LANE
        pad_e = _NSUB * eps - e
        ei = jnp.concatenate(
            [ei, jnp.full((2, pad_e), pad_node, jnp.int32)], axis=1)
    ei = ei.reshape(-1)

    x_flat = x[:, 0]
    ch = npad // _NSUB
    if (n % ch) % _LANE != 0:
        x_flat = jnp.concatenate(
            [x_flat, jnp.zeros((npad - n,), jnp.float32)])
        s = _sc_node_scalars(ei, x_flat, npad, npad, eps)
    else:
        s = _sc_node_scalars(ei, x_flat, n, npad, eps)

    return _tc_head(s.reshape(npad, 1), batch.reshape(n, 1), n, npad,
                    W1, b1, gamma, beta, y_extra,
                    W_l1[:h], W_l1[h:], b_l1, W_l2, b_l2)
